# Initial kernel scaffold; baseline (speedup 1.0000x reference)
import jax
import jax.numpy as jnp
from jax.experimental import pallas as pl


def kernel(x, edge_index, Wl1, Wr1, b1, g1, be1, Wl2, Wr2, b2, g2, be2, Wl3, Wr3, b3):
    raise NotImplementedError("write your pallas kernel here")


# speedup vs baseline: 7.2953x; 7.2953x over previous
import functools

import jax
import jax.numpy as jnp
from jax import lax
from jax.experimental import pallas as pl
from jax.experimental.pallas import tpu as pltpu
from jax.experimental.pallas import tpu_sc as plsc

N = 10000
E = 320000
NPAD = 10240
ROWS = 640
NC = 2
NS = 16
NW = NC * NS
EPW = E // NW
C = 80
NCH = EPW // C
BN = 1000
NB = N // BN
EPS = 1e-5

_f32 = jnp.float32


def _seg_sum_sc(D, with_cnt):
    mesh = plsc.VectorSubcoreMesh(
        core_axis_name="c", subcore_axis_name="s", num_cores=NC, num_subcores=NS)

    out_type = [jax.ShapeDtypeStruct((NC, NPAD, D), _f32)]
    if with_cnt:
        out_type.append(jax.ShapeDtypeStruct((NC, NPAD), _f32))

    scratch = [
        pltpu.VMEM_SHARED((NPAD, D), _f32),
        pltpu.VMEM((EPW,), jnp.int32),
        pltpu.VMEM((NCH, C), jnp.int32),
        pltpu.VMEM((C, D), _f32),
        pltpu.SemaphoreType.DMA,
    ]
    if with_cnt:
        scratch += [
            pltpu.VMEM_SHARED((NPAD,), _f32),
            pltpu.VMEM((C,), _f32),
        ]

    def body(*refs):
        if with_cnt:
            (t_hbm, src_hbm, dst_hbm, zrow_hbm, zcnt_hbm,
             out_hbm, cnt_hbm, acc_sh, src_v, dst_v, rows_v, sem,
             cnt_sh, ones_v) = refs
        else:
            (t_hbm, src_hbm, dst_hbm, zrow_hbm,
             out_hbm, acc_sh, src_v, dst_v, rows_v, sem) = refs
        c = lax.axis_index("c")
        s = lax.axis_index("s")
        wid = s * NC + c
        lo = pl.multiple_of(s * ROWS, 8)

        pltpu.sync_copy(zrow_hbm, acc_sh.at[pl.ds(lo, ROWS)])
        ebase = pl.multiple_of(wid * EPW, 8)
        pltpu.sync_copy(src_hbm.at[pl.ds(ebase, EPW)], src_v)
        pltpu.sync_copy(dst_hbm.at[wid], dst_v)
        if with_cnt:
            pltpu.sync_copy(zcnt_hbm, cnt_sh.at[pl.ds(lo, ROWS)])
            for k in range(C // 16):
                ones_v[pl.ds(k * 16, 16)] = jnp.ones((16,), _f32)
        plsc.subcore_barrier()

        def chunk(j, carry):
            off = pl.multiple_of(j * C, 8)
            pltpu.async_copy(
                t_hbm.at[src_v.at[pl.ds(off, C)]], rows_v, sem).wait()
            pltpu.sync_copy(rows_v, acc_sh.at[dst_v.at[j]], add=True)
            if with_cnt:
                pltpu.sync_copy(ones_v, cnt_sh.at[dst_v.at[j]], add=True)
            return carry

        lax.fori_loop(0, NCH, chunk, 0)
        plsc.subcore_barrier()

        pltpu.sync_copy(acc_sh.at[pl.ds(lo, ROWS)],
                        out_hbm.at[c].at[pl.ds(lo, ROWS)])
        if with_cnt:
            pltpu.sync_copy(cnt_sh.at[pl.ds(lo, ROWS)],
                            cnt_hbm.at[c].at[pl.ds(lo, ROWS)])

    return pl.kernel(body, out_type=tuple(out_type), mesh=mesh,
                     scratch_types=scratch)


def _pre_transform(x, wt):
    Din = x.shape[1]
    Dout = wt.shape[1]

    def body(x_ref, w_ref, o_ref):
        o_ref[...] = jnp.dot(x_ref[...], w_ref[...],
                             preferred_element_type=_f32)

    return pl.pallas_call(
        body,
        grid=(NB,),
        in_specs=[
            pl.BlockSpec((BN, Din), lambda i: (i, 0)),
            pl.BlockSpec((Din, Dout), lambda i: (0, 0)),
        ],
        out_specs=pl.BlockSpec((BN, Dout), lambda i: (i, 0)),
        out_shape=jax.ShapeDtypeStruct((N, Dout), _f32),
    )(x, wt)


def _combine_mid(s_part, invc, h, wrt, b2, gs2, be2, wlt_next):
    D = h.shape[1]
    Dn = wlt_next.shape[1]

    def body(s_ref, inv_ref, h_ref, wr_ref, b_ref, g_ref, be_ref, wl_ref,
             ho_ref, to_ref):
        st = s_ref[0] + s_ref[1]
        z = (st * inv_ref[0]
             + jnp.dot(h_ref[...], wr_ref[...], preferred_element_type=_f32)
             + b_ref[...])
        hh = jnp.maximum(z * g_ref[...] + be_ref[...], 0.0)
        ho_ref[...] = hh
        to_ref[...] = jnp.dot(hh, wl_ref[...], preferred_element_type=_f32)

    return pl.pallas_call(
        body,
        grid=(NB,),
        in_specs=[
            pl.BlockSpec((NC, BN, D), lambda i: (0, i, 0)),
            pl.BlockSpec((1, BN, 1), lambda i: (i, 0, 0)),
            pl.BlockSpec((BN, D), lambda i: (i, 0)),
            pl.BlockSpec((D, D), lambda i: (0, 0)),
            pl.BlockSpec((1, D), lambda i: (0, 0)),
            pl.BlockSpec((1, D), lambda i: (0, 0)),
            pl.BlockSpec((1, D), lambda i: (0, 0)),
            pl.BlockSpec((D, Dn), lambda i: (0, 0)),
        ],
        out_specs=[
            pl.BlockSpec((BN, D), lambda i: (i, 0)),
            pl.BlockSpec((BN, Dn), lambda i: (i, 0)),
        ],
        out_shape=[
            jax.ShapeDtypeStruct((N, D), _f32),
            jax.ShapeDtypeStruct((N, Dn), _f32),
        ],
    )(s_part, invc, h, wrt, b2, gs2, be2, wlt_next)


def _combine_out(s_part, invc, h, wlt, wrt, b2):
    D = h.shape[1]
    Do = wrt.shape[1]

    def body(s_ref, inv_ref, h_ref, wl_ref, wr_ref, b_ref, o_ref):
        agg = (s_ref[0] + s_ref[1]) * inv_ref[0]
        o_ref[...] = (jnp.dot(agg, wl_ref[...], preferred_element_type=_f32)
                      + jnp.dot(h_ref[...], wr_ref[...],
                                preferred_element_type=_f32)
                      + b_ref[...])

    return pl.pallas_call(
        body,
        grid=(NB,),
        in_specs=[
            pl.BlockSpec((NC, BN, D), lambda i: (0, i, 0)),
            pl.BlockSpec((1, BN, 1), lambda i: (i, 0, 0)),
            pl.BlockSpec((BN, D), lambda i: (i, 0)),
            pl.BlockSpec((D, Do), lambda i: (0, 0)),
            pl.BlockSpec((D, Do), lambda i: (0, 0)),
            pl.BlockSpec((1, Do), lambda i: (0, 0)),
        ],
        out_specs=pl.BlockSpec((BN, Do), lambda i: (i, 0)),
        out_shape=jax.ShapeDtypeStruct((N, Do), _f32),
    )(s_part, invc, h, wlt, wrt, b2)


def kernel(x, edge_index, Wl1, Wr1, b1, g1, be1, Wl2, Wr2, b2, g2, be2,
           Wl3, Wr3, b3):
    src = edge_index[0]
    dst3 = edge_index[1].reshape(NW, NCH, C)

    zrow128 = jnp.zeros((ROWS, 128), _f32)
    zcnt = jnp.zeros((ROWS,), _f32)

    bnscale = 1.0 / jnp.sqrt(jnp.float32(1.0 + EPS))

    seg128_cnt = _seg_sum_sc(128, True)
    seg128 = _seg_sum_sc(128, False)

    t1 = _pre_transform(x, Wl1.T)
    s1, cnt = seg128_cnt(t1, src, dst3, zrow128, zcnt)
    cnt_tot = cnt[0, :N] + cnt[1, :N]
    inv = 1.0 / jnp.clip(cnt_tot, 1.0, None)
    invc = inv.reshape(NB, BN, 1)
    h1, t2 = _combine_mid(s1, invc, x, Wr1.T, b1[None, :],
                          (g1 * bnscale)[None, :], be1[None, :], Wl2.T)

    (s2,) = seg128(t2, src, dst3, zrow128)
    h2, _ = _combine_mid(s2, invc, h1, Wr2.T, b2[None, :],
                         (g2 * bnscale)[None, :], be2[None, :], Wl3.T)

    (s3,) = seg128(h2, src, dst3, zrow128)
    out = _combine_out(s3, invc, h2, Wl3.T, Wr3.T, b3[None, :])
    return out
